# baseline (speedup 1.0000x reference)
import functools

import jax
import jax.numpy as jnp
from jax import lax
from jax.experimental import pallas as pl
from jax.experimental.pallas import tpu as pltpu

_ANCHORS = ((12.0, 16.0), (19.0, 36.0), (40.0, 28.0), (36.0, 75.0),
            (76.0, 55.0), (72.0, 146.0), (142.0, 110.0), (192.0, 243.0),
            (459.0, 401.0))
_STRIDES = (32, 16, 8)
_AMASKS = ((6, 7, 8), (3, 4, 5), (0, 1, 2))
_NCLS = 80
_NCH = 5 + _NCLS
_M = 20
_B = 16
_P = 24


def _prep_kernel(t_ref, o_ref):
    cls = t_ref[0]
    xn = t_ref[1]
    yn = t_ref[2]
    wn = t_ref[3]
    hn = t_ref[4]
    s = cls + xn + yn + wn + hn
    validrow = (s > 0.0).astype(jnp.float32)
    nlabel = jnp.sum(validrow, axis=1, keepdims=True)
    iota_m = lax.broadcasted_iota(jnp.int32, (_B, _M), 1).astype(jnp.float32)
    valid = (iota_m < nlabel).astype(jnp.float32)
    hasl = jnp.where(nlabel > 0.0, 1.0, 0.0) + jnp.zeros((_B, _M), jnp.float32)
    w5 = wn * 512.0
    h5 = hn * 512.0
    best = jnp.zeros((_B, _M), jnp.float32)
    cur = None
    for k in range(9):
        wa, ha = _ANCHORS[k]
        iw = jnp.minimum(w5, wa)
        ih = jnp.minimum(h5, ha)
        en = ((iw > 0.0) & (ih > 0.0)).astype(jnp.float32)
        ai = iw * ih * en
        iou = ai / (w5 * h5 + wa * ha - ai + 1e-16)
        if cur is None:
            cur = iou
        else:
            upd = iou > cur
            best = jnp.where(upd, float(k), best)
            cur = jnp.where(upd, iou, cur)
    a = best - 3.0 * jnp.floor(best / 3.0)
    blayer = jnp.floor(best / 3.0)
    wab = jnp.zeros_like(best)
    hab = jnp.zeros_like(best)
    for k in range(9):
        wab = jnp.where(best == float(k), _ANCHORS[k][0], wab)
        hab = jnp.where(best == float(k), _ANCHORS[k][1], hab)
    twlog = jnp.log(w5 / wab + 1e-16)
    thlog = jnp.log(h5 / hab + 1e-16)
    sc = jnp.sqrt(2.0 - wn * hn)
    o_ref[0] = valid
    o_ref[1] = hasl
    o_ref[2] = a
    o_ref[3] = blayer
    o_ref[4] = twlog
    o_ref[5] = thlog
    o_ref[6] = sc
    o_ref[7] = cls
    o_ref[8] = xn
    o_ref[9] = yn
    o_ref[10] = wn
    o_ref[11] = hn
    for l in range(3):
        f = float(512 // _STRIDES[l])
        il = jnp.floor(xn * f)
        jl = jnp.floor(yn * f)
        condl = (valid > 0.0) & (blayer == float(2 - l))
        clsact = jnp.zeros((_B, _M), jnp.float32)
        lw = jnp.zeros((_B, _M), jnp.float32)
        for m in range(_M):
            eqc = ((a == a[:, m:m + 1]) & (il == il[:, m:m + 1]) &
                   (jl == jl[:, m:m + 1]))
            eq = eqc & (cls == cls[:, m:m + 1])
            later = iota_m > float(m)
            dup = jnp.max(jnp.where(condl & eq & later, 1.0, 0.0),
                          axis=1, keepdims=True)
            dupc = jnp.max(jnp.where(condl & eqc & later, 1.0, 0.0),
                           axis=1, keepdims=True)
            sel = condl[:, m:m + 1]
            val = jnp.where(sel & (dup < 0.5), 1.0, 0.0)
            vlw = jnp.where(sel & (dupc < 0.5), 1.0, 0.0)
            clsact = jnp.where(iota_m == float(m), val, clsact)
            lw = jnp.where(iota_m == float(m), vlw, lw)
        o_ref[12 + l] = clsact
        o_ref[15 + l] = lw
    for p in range(18, _P):
        o_ref[p] = jnp.zeros((_B, _M), jnp.float32)


def _layer_kernel(prep_ref, x_ref, o_ref, csum_ref, *, lid, f):
    stride = _STRIDES[lid]
    was = [_ANCHORS[k][0] / stride for k in _AMASKS[lid]]
    has = [_ANCHORS[k][1] / stride for k in _AMASKS[lid]]
    iif = lax.broadcasted_iota(jnp.int32, (f, f), 1).astype(jnp.float32)
    jjf = lax.broadcasted_iota(jnp.int32, (f, f), 0).astype(jnp.float32)
    sx, sy, so, wrs, hrs = [], [], [], [], []
    px, py, pw, ph, pa, phw, phh = [], [], [], [], [], [], []
    pxl, pxr, pyt, pyb, thr = [], [], [], [], []
    mx, assigned = [], []
    txf, tyf, twl, thl, scv = [], [], [], [], []
    for anc in range(3):
        base = _NCH * anc
        xr = x_ref[0, base + 0]
        yr = x_ref[0, base + 1]
        wr = x_ref[0, base + 2]
        hr = x_ref[0, base + 3]
        obr = x_ref[0, base + 4]
        sx.append(jax.nn.sigmoid(xr))
        sy.append(jax.nn.sigmoid(yr))
        so.append(jax.nn.sigmoid(obr))
        wrs.append(wr)
        hrs.append(hr)
        px.append(sx[anc] + iif)
        py.append(sy[anc] + jjf)
        pw.append(jnp.exp(wr) * was[anc])
        ph.append(jnp.exp(hr) * has[anc])
        pa.append(pw[anc] * ph[anc])
        phw.append(pw[anc] * 0.5)
        phh.append(ph[anc] * 0.5)
        pxl.append(px[anc] - phw[anc])
        pxr.append(px[anc] + phw[anc])
        pyt.append(py[anc] - phh[anc])
        pyb.append(py[anc] + phh[anc])
        thr.append(0.7 * pa[anc])
        mx.append(jnp.zeros((f, f), jnp.bool_))
        assigned.append(jnp.zeros((f, f), jnp.bool_))
        txf.append(jnp.zeros((f, f), jnp.float32))
        tyf.append(jnp.zeros((f, f), jnp.float32))
        twl.append(jnp.zeros((f, f), jnp.float32))
        thl.append(jnp.zeros((f, f), jnp.float32))
        scv.append(jnp.zeros((f, f), jnp.float32))
    csum_ref[0] = 0.0
    for m in range(_M):
        valid = prep_ref[0, 0, m] > 0.0
        am = prep_ref[0, 2, m]
        bl = prep_ref[0, 3, m]
        twlog = prep_ref[0, 4, m]
        thlog = prep_ref[0, 5, m]
        scm = prep_ref[0, 6, m]
        tx = prep_ref[0, 8, m] * f
        ty = prep_ref[0, 9, m] * f
        tw = prep_ref[0, 10, m] * f
        th = prep_ref[0, 11, m] * f
        hw = tw * 0.5
        hh = th * 0.5
        im = jnp.floor(tx)
        jm = jnp.floor(ty)
        onlayer = valid & (bl == float(2 - lid))
        cellm = (iif == im) & (jjf == jm)
        txl = tx - hw
        txr = tx + hw
        tyt = ty - hh
        tyb = ty + hh
        cs = 0.7 * (tw * th + 1e-16)
        for anc in range(3):
            tlx = jnp.maximum(pxl[anc], txl)
            brx = jnp.minimum(pxr[anc], txr)
            tly = jnp.maximum(pyt[anc], tyt)
            bry = jnp.minimum(pyb[anc], tyb)
            en = (tlx < brx) & (tly < bry)
            ai = (brx - tlx) * (bry - tly)
            ign = valid & en & (1.7 * ai > thr[anc] + cs)
            mx[anc] = mx[anc] | ign
            condm = onlayer & (am == float(anc))
            mv = condm & cellm
            assigned[anc] = assigned[anc] | mv
            txf[anc] = jnp.where(mv, tx - im, txf[anc])
            tyf[anc] = jnp.where(mv, ty - jm, tyf[anc])
            twl[anc] = jnp.where(mv, twlog, twl[anc])
            thl[anc] = jnp.where(mv, thlog, thl[anc])
            scv[anc] = jnp.where(mv, scm, scv[anc])
        lwm = prep_ref[0, 15 + lid, m] > 0.0
        actm = prep_ref[0, 12 + lid, m] > 0.0
        cidx = prep_ref[0, 7, m].astype(jnp.int32)
        jm_i = jm.astype(jnp.int32)
        im_i = im.astype(jnp.int32)
        am_i = am.astype(jnp.int32)

        @pl.when(onlayer & (lwm | actm))
        def _():
            ch0 = am_i * _NCH + 5
            rows = x_ref[0, pl.ds(ch0, _NCLS), pl.ds(jm_i, 1), :][:, 0, :]
            lane_m = lax.broadcasted_iota(jnp.int32, (_NCLS, f), 1) == im_i
            vcol = jnp.sum(jnp.where(lane_m, rows, 0.0), axis=1,
                           keepdims=True)
            p = jnp.clip(jax.nn.sigmoid(vcol), 1e-7, 1.0 - 1e-7)
            nl1 = -jnp.log(1.0 - p)
            t_lw = jnp.sum(nl1)
            rowm = lax.broadcasted_iota(jnp.int32, (_NCLS, 1), 0) == cidx
            t_act = jnp.sum(jnp.where(rowm, -jnp.log(p) - nl1, 0.0))
            csum_ref[0] += (jnp.where(lwm, t_lw, 0.0) +
                            jnp.where(actm, t_act, 0.0))

    hasl = prep_ref[0, 1, 0] > 0.0
    c0 = -jnp.log(1.0 - jnp.clip(jnp.float32(0.0), 1e-7, 1.0 - 1e-7))
    total = csum_ref[0]
    for anc in range(3):
        asf = assigned[anc].astype(jnp.float32)
        omb = jnp.where(hasl & mx[anc], 0.0, 1.0)
        om = jnp.where(assigned[anc], 1.0, omb)
        pobj = jnp.clip(so[anc] * om, 1e-7, 1.0 - 1e-7)
        lobj = -(asf * jnp.log(pobj) + (1.0 - asf) * jnp.log(1.0 - pobj))
        w2 = scv[anc] * scv[anc]
        pxc = jnp.clip(sx[anc] * asf, 1e-7, 1.0 - 1e-7)
        pyc = jnp.clip(sy[anc] * asf, 1e-7, 1.0 - 1e-7)
        txt = txf[anc] * asf
        tyt = tyf[anc] * asf
        lxy = (-(txt * jnp.log(pxc) + (1.0 - txt) * jnp.log(1.0 - pxc)) * w2
               - (tyt * jnp.log(pyc) + (1.0 - tyt) * jnp.log(1.0 - pyc)) * w2)
        dw = wrs[anc] * asf * scv[anc] - twl[anc] * asf * scv[anc]
        dh = hrs[anc] * asf * scv[anc] - thl[anc] * asf * scv[anc]
        lwh = 0.5 * (dw * dw + dh * dh)
        lcls = jnp.where(assigned[anc], 0.0, _NCLS * c0)
        total = total + jnp.sum(lobj + lxy + lwh + lcls)
    ri = lax.broadcasted_iota(jnp.int32, (8, 128), 0)
    ci = lax.broadcasted_iota(jnp.int32, (8, 128), 1)
    o_ref[...] = jnp.where((ri == 0) & (ci == 0), total, 0.0)[None]


def _run_layer(prep, x, lid, f):
    kern = functools.partial(_layer_kernel, lid=lid, f=f)
    return pl.pallas_call(
        kern,
        grid=(_B,),
        in_specs=[
            pl.BlockSpec((1, _P, _M), lambda b: (b, 0, 0),
                         memory_space=pltpu.SMEM),
            pl.BlockSpec((1, 3 * _NCH, f, f), lambda b: (b, 0, 0, 0)),
        ],
        out_specs=pl.BlockSpec((1, 8, 128), lambda b: (b, 0, 0)),
        out_shape=jax.ShapeDtypeStruct((_B, 8, 128), jnp.float32),
        scratch_shapes=[pltpu.SMEM((1,), jnp.float32)],
        compiler_params=pltpu.CompilerParams(
            dimension_semantics=("parallel",)),
    )(prep, x)


def kernel(out0, out1, out2, targets):
    tgt_t = jnp.transpose(targets, (2, 0, 1))
    prep = pl.pallas_call(
        _prep_kernel,
        out_shape=jax.ShapeDtypeStruct((_P, _B, _M), jnp.float32),
    )(tgt_t)
    prep = jnp.transpose(prep, (1, 0, 2))
    total = jnp.float32(0.0)
    for lid, out in enumerate((out0, out1, out2)):
        f = out.shape[2]
        total = total + jnp.sum(_run_layer(prep, out, lid, f)[:, 0, 0])
    return total


# speedup vs baseline: 1.0477x; 1.0477x over previous
import functools

import jax
import jax.numpy as jnp
from jax import lax
from jax.experimental import pallas as pl
from jax.experimental.pallas import tpu as pltpu

_ANCHORS = ((12.0, 16.0), (19.0, 36.0), (40.0, 28.0), (36.0, 75.0),
            (76.0, 55.0), (72.0, 146.0), (142.0, 110.0), (192.0, 243.0),
            (459.0, 401.0))
_STRIDES = (32, 16, 8)
_AMASKS = ((6, 7, 8), (3, 4, 5), (0, 1, 2))
_NCLS = 80
_NCH = 5 + _NCLS
_M = 20
_B = 16
_P = 24


def _prep_kernel(t_ref, o_ref):
    cls = t_ref[0]
    xn = t_ref[1]
    yn = t_ref[2]
    wn = t_ref[3]
    hn = t_ref[4]
    s = cls + xn + yn + wn + hn
    validrow = (s > 0.0).astype(jnp.float32)
    nlabel = jnp.sum(validrow, axis=1, keepdims=True)
    iota_m = lax.broadcasted_iota(jnp.int32, (_B, _M), 1).astype(jnp.float32)
    valid = (iota_m < nlabel).astype(jnp.float32)
    hasl = jnp.where(nlabel > 0.0, 1.0, 0.0) + jnp.zeros((_B, _M), jnp.float32)
    w5 = wn * 512.0
    h5 = hn * 512.0
    best = jnp.zeros((_B, _M), jnp.float32)
    cur = None
    for k in range(9):
        wa, ha = _ANCHORS[k]
        iw = jnp.minimum(w5, wa)
        ih = jnp.minimum(h5, ha)
        en = ((iw > 0.0) & (ih > 0.0)).astype(jnp.float32)
        ai = iw * ih * en
        iou = ai / (w5 * h5 + wa * ha - ai + 1e-16)
        if cur is None:
            cur = iou
        else:
            upd = iou > cur
            best = jnp.where(upd, float(k), best)
            cur = jnp.where(upd, iou, cur)
    a = best - 3.0 * jnp.floor(best / 3.0)
    blayer = jnp.floor(best / 3.0)
    wab = jnp.zeros_like(best)
    hab = jnp.zeros_like(best)
    for k in range(9):
        wab = jnp.where(best == float(k), _ANCHORS[k][0], wab)
        hab = jnp.where(best == float(k), _ANCHORS[k][1], hab)
    twlog = jnp.log(w5 / wab + 1e-16)
    thlog = jnp.log(h5 / hab + 1e-16)
    sc = jnp.sqrt(2.0 - wn * hn)
    o_ref[0] = valid
    o_ref[1] = hasl
    o_ref[2] = a
    o_ref[3] = blayer
    o_ref[4] = twlog
    o_ref[5] = thlog
    o_ref[6] = sc
    o_ref[7] = cls
    o_ref[8] = xn
    o_ref[9] = yn
    o_ref[10] = wn
    o_ref[11] = hn
    for l in range(3):
        f = float(512 // _STRIDES[l])
        il = jnp.floor(xn * f)
        jl = jnp.floor(yn * f)
        condl = (valid > 0.0) & (blayer == float(2 - l))
        clsact = jnp.zeros((_B, _M), jnp.float32)
        lw = jnp.zeros((_B, _M), jnp.float32)
        for m in range(_M):
            eqc = ((a == a[:, m:m + 1]) & (il == il[:, m:m + 1]) &
                   (jl == jl[:, m:m + 1]))
            eq = eqc & (cls == cls[:, m:m + 1])
            later = iota_m > float(m)
            dup = jnp.max(jnp.where(condl & eq & later, 1.0, 0.0),
                          axis=1, keepdims=True)
            dupc = jnp.max(jnp.where(condl & eqc & later, 1.0, 0.0),
                           axis=1, keepdims=True)
            sel = condl[:, m:m + 1]
            val = jnp.where(sel & (dup < 0.5), 1.0, 0.0)
            vlw = jnp.where(sel & (dupc < 0.5), 1.0, 0.0)
            clsact = jnp.where(iota_m == float(m), val, clsact)
            lw = jnp.where(iota_m == float(m), vlw, lw)
        o_ref[12 + l] = clsact
        o_ref[15 + l] = lw
    for p in range(18, _P):
        o_ref[p] = jnp.zeros((_B, _M), jnp.float32)


def _merged_kernel(prep_ref, x0_ref, x1_ref, x2_ref, o_ref, csum_ref):
    x_refs = (x0_ref, x1_ref, x2_ref)
    fs = (16, 32, 64)
    sx = {}
    sy = {}
    so = {}
    wrs = {}
    hrs = {}
    pxl = {}
    pxr = {}
    pyt = {}
    pyb = {}
    thr = {}
    iif = {}
    jjf = {}
    mx = {}
    assigned = {}
    txf = {}
    tyf = {}
    twl = {}
    thl = {}
    scv = {}
    for lid in range(3):
        f = fs[lid]
        stride = _STRIDES[lid]
        x_ref = x_refs[lid]
        iif[lid] = lax.broadcasted_iota(jnp.int32, (f, f), 1).astype(jnp.float32)
        jjf[lid] = lax.broadcasted_iota(jnp.int32, (f, f), 0).astype(jnp.float32)
        for anc in range(3):
            wa = _ANCHORS[_AMASKS[lid][anc]][0] / stride
            ha = _ANCHORS[_AMASKS[lid][anc]][1] / stride
            base = _NCH * anc
            k = (lid, anc)
            sx[k] = jax.nn.sigmoid(x_ref[0, base + 0])
            sy[k] = jax.nn.sigmoid(x_ref[0, base + 1])
            wr = x_ref[0, base + 2]
            hr = x_ref[0, base + 3]
            so[k] = jax.nn.sigmoid(x_ref[0, base + 4])
            wrs[k] = wr
            hrs[k] = hr
            pxv = sx[k] + iif[lid]
            pyv = sy[k] + jjf[lid]
            pwv = jnp.exp(wr) * wa
            phv = jnp.exp(hr) * ha
            pxl[k] = pxv - pwv * 0.5
            pxr[k] = pxv + pwv * 0.5
            pyt[k] = pyv - phv * 0.5
            pyb[k] = pyv + phv * 0.5
            thr[k] = 0.7 * (pwv * phv)
            mx[k] = jnp.zeros((f, f), jnp.bool_)
            assigned[k] = jnp.zeros((f, f), jnp.bool_)
            txf[k] = jnp.zeros((f, f), jnp.float32)
            tyf[k] = jnp.zeros((f, f), jnp.float32)
            twl[k] = jnp.zeros((f, f), jnp.float32)
            thl[k] = jnp.zeros((f, f), jnp.float32)
            scv[k] = jnp.zeros((f, f), jnp.float32)
    csum_ref[0] = 0.0
    for m in range(_M):
        valid = prep_ref[0, 0, m] > 0.0
        am = prep_ref[0, 2, m]
        bl = prep_ref[0, 3, m]
        twlog = prep_ref[0, 4, m]
        thlog = prep_ref[0, 5, m]
        scm = prep_ref[0, 6, m]
        cidx = prep_ref[0, 7, m].astype(jnp.int32)
        xn = prep_ref[0, 8, m]
        yn = prep_ref[0, 9, m]
        wn = prep_ref[0, 10, m]
        hn = prep_ref[0, 11, m]
        am_i = am.astype(jnp.int32)
        for lid in range(3):
            f = fs[lid]
            x_ref = x_refs[lid]
            tx = xn * f
            ty = yn * f
            tw = wn * f
            th = hn * f
            hw = tw * 0.5
            hh = th * 0.5
            im = jnp.floor(tx)
            jm = jnp.floor(ty)
            onlayer = valid & (bl == float(2 - lid))
            cellm = (iif[lid] == im) & (jjf[lid] == jm)
            txl = tx - hw
            txr = tx + hw
            tyt = ty - hh
            tyb = ty + hh
            cs = 0.7 * (tw * th + 1e-16)
            for anc in range(3):
                k = (lid, anc)
                tlx = jnp.maximum(pxl[k], txl)
                brx = jnp.minimum(pxr[k], txr)
                tly = jnp.maximum(pyt[k], tyt)
                bry = jnp.minimum(pyb[k], tyb)
                en = (tlx < brx) & (tly < bry)
                ai = (brx - tlx) * (bry - tly)
                ign = valid & en & (1.7 * ai > thr[k] + cs)
                mx[k] = mx[k] | ign
                condm = onlayer & (am == float(anc))
                mv = condm & cellm
                assigned[k] = assigned[k] | mv
                txf[k] = jnp.where(mv, tx - im, txf[k])
                tyf[k] = jnp.where(mv, ty - jm, tyf[k])
                twl[k] = jnp.where(mv, twlog, twl[k])
                thl[k] = jnp.where(mv, thlog, thl[k])
                scv[k] = jnp.where(mv, scm, scv[k])
            lwm = prep_ref[0, 15 + lid, m] > 0.0
            actm = prep_ref[0, 12 + lid, m] > 0.0
            jm_i = jm.astype(jnp.int32)
            im_i = im.astype(jnp.int32)

            @pl.when(onlayer & (lwm | actm))
            def _(x_ref=x_ref, jm_i=jm_i, im_i=im_i, f=f, lwm=lwm,
                  actm=actm, cidx=cidx, am_i=am_i):
                ch0 = am_i * _NCH + 5
                rows = x_ref[0, pl.ds(ch0, _NCLS), pl.ds(jm_i, 1), :][:, 0, :]
                lane_m = (lax.broadcasted_iota(jnp.int32, (_NCLS, f), 1)
                          == im_i)
                vcol = jnp.sum(jnp.where(lane_m, rows, 0.0), axis=1,
                               keepdims=True)
                p = jnp.clip(jax.nn.sigmoid(vcol), 1e-7, 1.0 - 1e-7)
                nl1 = -jnp.log(1.0 - p)
                t_lw = jnp.sum(nl1)
                rowm = (lax.broadcasted_iota(jnp.int32, (_NCLS, 1), 0)
                        == cidx)
                t_act = jnp.sum(jnp.where(rowm, -jnp.log(p) - nl1, 0.0))
                csum_ref[0] += (jnp.where(lwm, t_lw, 0.0) +
                                jnp.where(actm, t_act, 0.0))

    hasl = prep_ref[0, 1, 0] > 0.0
    c0 = -jnp.log(1.0 - jnp.clip(jnp.float32(0.0), 1e-7, 1.0 - 1e-7))
    total = csum_ref[0]
    for lid in range(3):
        for anc in range(3):
            k = (lid, anc)
            asf = assigned[k].astype(jnp.float32)
            omb = jnp.where(hasl & mx[k], 0.0, 1.0)
            om = jnp.where(assigned[k], 1.0, omb)
            pobj = jnp.clip(so[k] * om, 1e-7, 1.0 - 1e-7)
            lobj = -(asf * jnp.log(pobj) +
                     (1.0 - asf) * jnp.log(1.0 - pobj))
            w2 = scv[k] * scv[k]
            pxc = jnp.clip(sx[k] * asf, 1e-7, 1.0 - 1e-7)
            pyc = jnp.clip(sy[k] * asf, 1e-7, 1.0 - 1e-7)
            txt = txf[k] * asf
            tyt = tyf[k] * asf
            lxy = (-(txt * jnp.log(pxc) +
                     (1.0 - txt) * jnp.log(1.0 - pxc)) * w2
                   - (tyt * jnp.log(pyc) +
                      (1.0 - tyt) * jnp.log(1.0 - pyc)) * w2)
            dw = wrs[k] * asf * scv[k] - twl[k] * asf * scv[k]
            dh = hrs[k] * asf * scv[k] - thl[k] * asf * scv[k]
            lwh = 0.5 * (dw * dw + dh * dh)
            lcls = jnp.where(assigned[k], 0.0, _NCLS * c0)
            total = total + jnp.sum(lobj + lxy + lwh + lcls)
    ri = lax.broadcasted_iota(jnp.int32, (8, 128), 0)
    ci = lax.broadcasted_iota(jnp.int32, (8, 128), 1)
    o_ref[...] = jnp.where((ri == 0) & (ci == 0), total, 0.0)[None]


def _run_merged(prep, x0, x1, x2):
    return pl.pallas_call(
        _merged_kernel,
        grid=(_B,),
        in_specs=[
            pl.BlockSpec((1, _P, _M), lambda b: (b, 0, 0),
                         memory_space=pltpu.SMEM),
            pl.BlockSpec((1, 3 * _NCH, 16, 16), lambda b: (b, 0, 0, 0)),
            pl.BlockSpec((1, 3 * _NCH, 32, 32), lambda b: (b, 0, 0, 0)),
            pl.BlockSpec((1, 3 * _NCH, 64, 64), lambda b: (b, 0, 0, 0)),
        ],
        out_specs=pl.BlockSpec((1, 8, 128), lambda b: (b, 0, 0)),
        out_shape=jax.ShapeDtypeStruct((_B, 8, 128), jnp.float32),
        scratch_shapes=[pltpu.SMEM((1,), jnp.float32)],
        compiler_params=pltpu.CompilerParams(
            dimension_semantics=("parallel",)),
    )(prep, x0, x1, x2)


def kernel(out0, out1, out2, targets):
    tgt_t = jnp.transpose(targets, (2, 0, 1))
    prep = pl.pallas_call(
        _prep_kernel,
        out_shape=jax.ShapeDtypeStruct((_P, _B, _M), jnp.float32),
    )(tgt_t)
    prep = jnp.transpose(prep, (1, 0, 2))
    return jnp.sum(_run_merged(prep, out0, out1, out2)[:, 0, 0])
